# Initial kernel scaffold; baseline (speedup 1.0000x reference)
import jax
import jax.numpy as jnp
from jax.experimental import pallas as pl


def kernel(scores, batch_indices, boxes):
    raise NotImplementedError("write your pallas kernel here")


# speedup vs baseline: 12.8830x; 12.8830x over previous
import functools

import jax
import jax.numpy as jnp
from jax import lax
from jax.experimental import pallas as pl
from jax.experimental.pallas import tpu as pltpu
from jax.experimental.pallas import tpu_sc as plsc

N = 320000
C = 80
NB = 16
K = 1000
L = 16
NBKT = 2048
NGRP = NBKT // L
CAP_A = 1024
CAP_B = 256
CHUNK = 8192
FPAD = 24
SHARD = N // 16
BOUT = 1008

_mesh = plsc.VectorSubcoreMesh(core_axis_name="c", subcore_axis_name="s")
_cparams = pltpu.CompilerParams(
    needs_layout_passes=False, use_tc_tiling_on_sc=False)


def _scalar(x):
    return jnp.max(x) if getattr(x, "ndim", 0) else x


def _block_sort_desc(kref, vref, nblocks):
    def presort(i, _):
        sk, sv = plsc.sort_key_val(
            kref[pl.ds(i * L, L)], vref[pl.ds(i * L, L)], descending=True)
        kref[pl.ds(i * L, L)] = sk
        vref[pl.ds(i * L, L)] = sv
        return 0

    lax.fori_loop(0, nblocks, presort, 0)

    kk = 2
    while kk <= nblocks:
        j = kk // 2
        while j >= 1:
            def merge(i, _, j=j, kk=kk):
                part = i ^ j

                @pl.when(part > i)
                def _():
                    ak = kref[pl.ds(i * L, L)]
                    av = vref[pl.ds(i * L, L)]
                    bk = lax.rev(kref[pl.ds(part * L, L)], (0,))
                    bv = lax.rev(vref[pl.ds(part * L, L)], (0,))
                    cmp = ak >= bk
                    hik = jnp.where(cmp, ak, bk)
                    hiv = jnp.where(cmp, av, bv)
                    lok = jnp.where(cmp, bk, ak)
                    lov = jnp.where(cmp, bv, av)
                    hik, hiv = plsc.sort_key_val(hik, hiv, descending=True)
                    lok, lov = plsc.sort_key_val(lok, lov, descending=True)
                    desc_i = (i & kk) == 0
                    kref[pl.ds(i * L, L)] = jnp.where(desc_i, hik, lok)
                    vref[pl.ds(i * L, L)] = jnp.where(desc_i, hiv, lov)
                    kref[pl.ds(part * L, L)] = jnp.where(desc_i, lok, hik)
                    vref[pl.ds(part * L, L)] = jnp.where(desc_i, lov, hiv)

                return 0

            lax.fori_loop(0, nblocks, merge, 0)
            j //= 2
        kk *= 2


def _tie_fix(kref, vref, vtmp, n, lanes, passes=4):
    for p in range(passes):
        par = p % 2

        def tpass(t, _, par=par):
            pos = t * L + lanes
            left = (pos & 1) == par
            prt = pos + jnp.where(left, 1, -1)
            ok = (prt >= 0) & (prt < n)
            prt_c = jnp.clip(prt, 0, n - 1)
            myk = kref[pl.ds(t * L, L)]
            myv = vref[pl.ds(t * L, L)]
            pk = plsc.load_gather(kref, [prt_c])
            pv = plsc.load_gather(vref, [prt_c])
            eq = (myk == pk) & ok
            swap = eq & jnp.where(left, myv > pv, pv > myv)
            vtmp[pl.ds(t * L, L)] = jnp.where(swap, pv, myv)
            return 0

        lax.fori_loop(0, n // L, tpass, 0)

        def copy_back(t, _):
            vref[pl.ds(t * L, L)] = vtmp[pl.ds(t * L, L)]
            return 0

        lax.fori_loop(0, n // L, copy_back, 0)


def _k1_body(u_hbm, bidx, top_idx_out, top_batch_out,
             shard_buf, bh, grid_l, counts_v, hist, totals,
             fbuf, ka, va, kb, vb, vtmp, topv, bout, grid_sh, sem):
    c = lax.axis_index("c")
    s = lax.axis_index("s")
    lanes = lax.iota(jnp.int32, L)
    zeros16 = jnp.zeros((L,), jnp.int32)
    ones16 = jnp.ones((L,), jnp.int32)

    for g in range(NB):
        bh[pl.ds(g * L, L)] = zeros16
    pltpu.sync_copy(bidx.at[pl.ds(s * SHARD, SHARD)], shard_buf)
    laneoff_b = lanes * NB

    def bcount(t, _):
        v = shard_buf[pl.ds(t * L, L)]
        plsc.addupdate_scatter(bh, [v + laneoff_b], ones16)
        return 0

    lax.fori_loop(0, SHARD // L, bcount, 0)
    acc = zeros16
    for l in range(L):
        acc = acc + bh[pl.ds(l * NB, L)]
    counts_v[pl.ds(0, L)] = acc
    pltpu.sync_copy(counts_v, grid_sh.at[pl.ds(s * NB, NB)])
    plsc.subcore_barrier()
    pltpu.sync_copy(grid_sh, grid_l)
    tot = zeros16
    for w in range(16):
        tot = tot + grid_l[pl.ds(w * NB, L)]
    starts_ex = plsc.cumsum(tot) - tot

    b = c * 8 + s
    bm = lanes == b
    start = jnp.max(jnp.where(bm, starts_ex, 0))
    m = jnp.max(jnp.where(bm, tot, 0))

    def load_chunk(ci):
        astart = start + ci * CHUNK
        abase = jnp.minimum((astart // 8) * 8, N - CHUNK - 8)
        pltpu.sync_copy(u_hbm.at[pl.ds(abase, CHUNK + 8)],
                        fbuf.at[pl.ds(0, CHUNK + 8)])
        return astart - abase

    @pl.when(s < 8)
    def _selection():
        zf = jnp.zeros((L,), jnp.float32)
        for t in range(CAP_A // L):
            ka[pl.ds(t * L, L)] = zf
            va[pl.ds(t * L, L)] = zeros16
            vtmp[pl.ds(t * L, L)] = zeros16
            topv[pl.ds(t * L, L)] = zeros16
        for t in range(CAP_B // L):
            kb[pl.ds(t * L, L)] = zf
            vb[pl.ds(t * L, L)] = zeros16
        bvec = zeros16 + b
        for t in range(BOUT // L):
            bout[pl.ds(t * L, L)] = bvec

        @pl.when(m > 0)
        def _nonempty():
            def zh(t, _):
                hist[pl.ds(t * L, L)] = zeros16
                return 0

            lax.fori_loop(0, (NBKT * L) // L, zh, 0)
            laneoff_h = lanes * NBKT
            nchunks = (m + CHUNK - 1) // CHUNK

            def hchunk(ci, _):
                d = load_chunk(ci)
                cbase = ci * CHUNK

                def hb(t, _):
                    u = fbuf[pl.ds(d + t * L, L)]
                    f = 1.0 - u
                    bkt = jnp.minimum(
                        (f * float(NBKT)).astype(jnp.int32), NBKT - 1)
                    j = cbase + t * L + lanes
                    plsc.addupdate_scatter(
                        hist, [bkt + laneoff_h], ones16, mask=j < m)
                    return 0

                lax.fori_loop(0, CHUNK // L, hb, 0)
                return 0

            lax.fori_loop(0, nchunks, hchunk, 0)

            def lred(g, _):
                a2 = zeros16
                for l in range(L):
                    a2 = a2 + hist[pl.ds(l * NBKT + g * L, L)]
                totals[pl.ds(g * L, L)] = a2
                return 0

            lax.fori_loop(0, NGRP, lred, 0)

            def bscan(t, carry):
                csum, bstar, c1 = carry
                g = NGRP - 1 - t
                tv = totals[pl.ds(g * L, L)]
                rv = lax.rev(tv, (0,))
                cs = plsc.cumsum(rv)
                cross = (csum + cs) >= K
                nset = jnp.sum(cross.astype(jnp.int32))
                fl = L - nset
                found = (nset > 0) & (bstar < 0)
                excl = jnp.max(jnp.where(lanes == fl, cs - rv, 0))
                bkt_cand = g * L + (L - 1) - fl
                bstar = jnp.where(found, bkt_cand, bstar)
                c1 = jnp.where(found, csum + excl, c1)
                return (csum + jnp.sum(tv), bstar, c1)

            msum, bstar, c1 = lax.fori_loop(
                0, NGRP, bscan, (jnp.int32(0), jnp.int32(-1), jnp.int32(0)))
            t0 = jnp.sum(jnp.where(lanes == 0, totals[pl.ds(0, L)], 0))
            c1 = jnp.where(bstar < 0, msum - t0, c1)
            bstar = jnp.where(bstar < 0, 0, bstar)
            r = K - c1

            def cchunk(ci, carry):
                d = load_chunk(ci)
                cbase = ci * CHUNK

                def cb(t, c2):
                    pa, pb = c2
                    u = fbuf[pl.ds(d + t * L, L)]
                    f = 1.0 - u
                    bkt = jnp.minimum(
                        (f * float(NBKT)).astype(jnp.int32), NBKT - 1)
                    j = cbase + t * L + lanes
                    valid = j < m
                    gi = start + j
                    ma = (bkt > bstar) & valid & (pa < CAP_A - L)
                    mb = (bkt == bstar) & valid & (pb < CAP_B - L)
                    plsc.store_compressed(ka.at[pl.ds(pa, L)], f, mask=ma)
                    plsc.store_compressed(va.at[pl.ds(pa, L)], gi, mask=ma)
                    plsc.store_compressed(kb.at[pl.ds(pb, L)], f, mask=mb)
                    plsc.store_compressed(vb.at[pl.ds(pb, L)], gi, mask=mb)
                    pa = pa + _scalar(plsc.all_reduce_population_count(ma))
                    pb = pb + _scalar(plsc.all_reduce_population_count(mb))
                    return (pa, pb)

                return lax.fori_loop(0, CHUNK // L, cb, carry)

            lax.fori_loop(0, nchunks, cchunk, (jnp.int32(0), jnp.int32(0)))

            _block_sort_desc(ka, va, CAP_A // L)
            _tie_fix(ka, va, vtmp, CAP_A, lanes)
            _block_sort_desc(kb, vb, CAP_B // L)
            _tie_fix(kb, vb, vtmp, CAP_B, lanes)

            def asm_a(t, _):
                pos = t * L + lanes
                cur = topv[pl.ds(t * L, L)]
                topv[pl.ds(t * L, L)] = jnp.where(
                    pos < c1, va[pl.ds(t * L, L)], cur)
                return 0

            lax.fori_loop(0, CAP_A // L, asm_a, 0)

            def asm_b(t, _):
                jj = t * L + lanes
                tgt = jnp.minimum(c1 + jj, CAP_A - 1)
                plsc.store_scatter(
                    topv, [tgt], vb[pl.ds(t * L, L)], mask=jj < r)
                return 0

            lax.fori_loop(0, CAP_B // L, asm_b, 0)

        pltpu.sync_copy(topv, top_idx_out.at[pl.ds(b * CAP_A, CAP_A)])
        pltpu.sync_copy(bout.at[pl.ds(0, K)],
                        top_batch_out.at[pl.ds(b * K, K)])


_k1 = functools.partial(
    pl.kernel,
    out_type=(
        jax.ShapeDtypeStruct((NB * CAP_A,), jnp.int32),
        jax.ShapeDtypeStruct((NB * K,), jnp.int32),
    ),
    mesh=_mesh,
    compiler_params=_cparams,
    scratch_types=[
        pltpu.VMEM((SHARD,), jnp.int32),
        pltpu.VMEM((NB * L,), jnp.int32),
        pltpu.VMEM((16 * NB,), jnp.int32),
        pltpu.VMEM((L,), jnp.int32),
        pltpu.VMEM((NBKT * L,), jnp.int32),
        pltpu.VMEM((NBKT,), jnp.int32),
        pltpu.VMEM((CHUNK + FPAD,), jnp.float32),
        pltpu.VMEM((CAP_A,), jnp.float32),
        pltpu.VMEM((CAP_A,), jnp.int32),
        pltpu.VMEM((CAP_B,), jnp.float32),
        pltpu.VMEM((CAP_B,), jnp.int32),
        pltpu.VMEM((CAP_A,), jnp.int32),
        pltpu.VMEM((CAP_A,), jnp.int32),
        pltpu.VMEM((BOUT,), jnp.int32),
        pltpu.VMEM_SHARED((16 * NB,), jnp.int32),
        pltpu.SemaphoreType.DMA,
    ],
)(_k1_body)


def _k2_body(scores, boxes_flat, top_idx, fs_out, fb_flat_out,
             topb, bidx4, sbuf, bbuf, sem):
    c = lax.axis_index("c")
    s = lax.axis_index("s")
    lanes = lax.iota(jnp.int32, L)
    b = c * 8 + s

    @pl.when(s < 8)
    def _():
        pltpu.sync_copy(top_idx.at[pl.ds(b * CAP_A, CAP_A)], topb)
        for off, sz in ((0, 256), (256, 256), (512, 256), (768, 232)):
            pltpu.async_copy(
                scores.at[topb.at[pl.ds(off, 256)]], sbuf, sem).wait()
            pltpu.sync_copy(sbuf.at[pl.ds(0, sz)],
                            fs_out.at[pl.ds(b * K + off, sz)])
            def bi(t, _):
                rows = plsc.load_gather(topb, [off + t * 4 + lanes // 4])
                bidx4[pl.ds(t * L, L)] = rows * 4 + (lanes & 3)
                return 0

            lax.fori_loop(0, 64, bi, 0)
            pltpu.async_copy(boxes_flat.at[bidx4], bbuf, sem).wait()
            pltpu.sync_copy(bbuf.at[pl.ds(0, sz * 4)],
                            fb_flat_out.at[pl.ds((b * K + off) * 4, sz * 4)])


_k2 = functools.partial(
    pl.kernel,
    out_type=(
        jax.ShapeDtypeStruct((NB * K, C), jnp.float32),
        jax.ShapeDtypeStruct((NB * K * 4,), jnp.float32),
    ),
    mesh=_mesh,
    compiler_params=_cparams,
    scratch_types=[
        pltpu.VMEM((CAP_A,), jnp.int32),
        pltpu.VMEM((1024,), jnp.int32),
        pltpu.VMEM((256, C), jnp.float32),
        pltpu.VMEM((1024,), jnp.float32),
        pltpu.SemaphoreType.DMA,
    ],
)(_k2_body)


def kernel(scores, batch_indices, boxes):
    bidx = batch_indices.astype(jnp.int32)
    u = lax.slice_in_dim(scores, 0, 1, axis=1).reshape(-1)
    top_idx, top_batch = _k1(u, bidx)
    filtered_scores, fb_flat = _k2(scores, boxes.reshape(-1), top_idx)
    return (filtered_scores,
            top_batch.astype(batch_indices.dtype),
            fb_flat.reshape(NB * K, 4))
